# Initial kernel scaffold; baseline (speedup 1.0000x reference)
import jax
import jax.numpy as jnp
from jax.experimental import pallas as pl


def kernel(data, params):
    raise NotImplementedError("write your pallas kernel here")


# speedup vs baseline: 7.2059x; 7.2059x over previous
import functools

import numpy as np
import jax
import jax.numpy as jnp
from jax import lax
from jax.experimental import pallas as pl
from jax.experimental.pallas import tpu as pltpu
from jax.experimental.pallas import tpu_sc as plsc

F32 = jnp.float32
_K = 16
_HI = lax.Precision.HIGHEST
_DIMS = (32, 64, 128, 256, 512)


def _dotT(x, w):
    return lax.dot_general(x, w, (((1,), (1,)), ((), ())),
                           precision=_HI, preferred_element_type=F32)


def _dot(x, w):
    return lax.dot_general(x, w, (((1,), (0,)), ((), ())),
                           precision=_HI, preferred_element_type=F32)


def _mm_kernel(x_ref, w_ref, b_ref, y_ref, s1_ref, s2_ref):
    y = _dotT(x_ref[...], w_ref[...]) + b_ref[...]
    y_ref[...] = y
    s1_ref[0] = jnp.sum(y, axis=0, keepdims=True)
    s2_ref[0] = jnp.sum(y * y, axis=0, keepdims=True)


def _bn_kernel(y_ref, s1_ref, s2_ref, g_ref, b_ref, o_ref, *, n, cout):
    mu = jnp.sum(s1_ref[...], axis=0) / n
    ex2 = jnp.sum(s2_ref[...], axis=0) / n
    var = ex2 - mu * mu
    y = y_ref[...]
    o_ref[:, 0:cout] = jnp.maximum(
        (y - mu) / jnp.sqrt(var + 1e-5) * g_ref[...] + b_ref[...], 0.0)


def _lin_bn_relu(x, lin, bn, pad_to=None):
    n, cin = x.shape
    cout = lin['W'].shape[0]
    P = pad_to if pad_to is not None else cout
    T = min(n, 2048)
    G = n // T
    y, s1, s2 = pl.pallas_call(
        _mm_kernel,
        grid=(G,),
        in_specs=[pl.BlockSpec((T, cin), lambda i: (i, 0)),
                  pl.BlockSpec((cout, cin), lambda i: (0, 0)),
                  pl.BlockSpec((1, cout), lambda i: (0, 0))],
        out_specs=[pl.BlockSpec((T, cout), lambda i: (i, 0)),
                   pl.BlockSpec((1, 1, cout), lambda i: (i, 0, 0)),
                   pl.BlockSpec((1, 1, cout), lambda i: (i, 0, 0))],
        out_shape=[jax.ShapeDtypeStruct((n, cout), F32),
                   jax.ShapeDtypeStruct((G, 1, cout), F32),
                   jax.ShapeDtypeStruct((G, 1, cout), F32)],
    )(x, lin['W'], lin['b'].reshape(1, -1))
    return pl.pallas_call(
        functools.partial(_bn_kernel, n=float(n), cout=cout),
        grid=(G,),
        in_specs=[pl.BlockSpec((T, cout), lambda i: (i, 0)),
                  pl.BlockSpec((G, 1, cout), lambda i: (0, 0, 0)),
                  pl.BlockSpec((G, 1, cout), lambda i: (0, 0, 0)),
                  pl.BlockSpec((1, cout), lambda i: (0, 0)),
                  pl.BlockSpec((1, cout), lambda i: (0, 0))],
        out_specs=pl.BlockSpec((T, P), lambda i: (i, 0)),
        out_shape=jax.ShapeDtypeStruct((n, P), F32),
    )(y, s1, s2, bn['gamma'].reshape(1, -1), bn['beta'].reshape(1, -1))


def _fps_kernel(pt_ref, sub_ref, *, m, n):
    B = pt_ref.shape[0]
    pt = pt_ref[...]
    for j in range(3):
        sub_ref[:, 0:1, j:j + 1] = pt[:, j:j + 1, 0:1]
    iota = lax.broadcasted_iota(jnp.int32, (B, n), 1)

    def body(i, carry):
        dists, last = carry
        d = jnp.sum((pt - last) ** 2, axis=1)
        dists = jnp.minimum(dists, d)
        mx = jnp.max(dists, axis=1, keepdims=True)
        idx = jnp.min(jnp.where(dists == mx, iota, n), axis=1, keepdims=True)
        mask = (iota == idx)[:, None, :]
        lastc = jnp.sum(jnp.where(mask, pt, 0.0), axis=2, keepdims=True)
        for j in range(3):
            sub_ref[:, pl.ds(i, 1), j:j + 1] = lastc[:, j:j + 1, :]
        return dists, lastc

    init = (jnp.full((B, n), jnp.inf, F32), pt[:, :, 0:1])
    lax.fori_loop(1, m, body, init)


def _fps(posT, m):
    B, _, n = posT.shape
    return pl.pallas_call(
        functools.partial(_fps_kernel, m=m, n=n),
        grid=(1,),
        in_specs=[pl.BlockSpec((B, 3, n), lambda i: (0, 0, 0))],
        out_specs=pl.BlockSpec((B, m, 3), lambda i: (0, 0, 0)),
        out_shape=jax.ShapeDtypeStruct((B, m, 3), F32),
    )(posT)


def _knn_kernel(q_ref, st_ref, o_ref, *, ns, Tq, exclude_self):
    b = pl.program_id(0)
    t = pl.program_id(1)
    q = q_ref[0]
    st = st_ref[0]
    sq_q = jnp.sum(q * q, axis=1, keepdims=True)
    sq_s = jnp.sum(st * st, axis=0, keepdims=True)
    mm = lax.dot_general(q.astype(jnp.bfloat16), st.astype(jnp.bfloat16),
                         (((1,), (0,)), ((), ())),
                         preferred_element_type=F32)
    d = sq_q + sq_s - 2.0 * mm
    ci = lax.broadcasted_iota(jnp.int32, (Tq, ns), 1)
    if exclude_self:
        qid = t * Tq + lax.broadcasted_iota(jnp.int32, (Tq, ns), 0)
        d = jnp.where(ci == qid, jnp.inf, d)
    off = b * ns

    def step(k, dcur):
        mn = jnp.min(dcur, axis=1, keepdims=True)
        idx = jnp.min(jnp.where(dcur == mn, ci, ns), axis=1)
        o_ref[0, pl.ds(k, 1), :] = (idx + off)[None, :]
        return jnp.where(ci == idx[:, None], jnp.inf, dcur)

    lax.fori_loop(0, _K, step, d)


def _knn(qpos, sposT, exclude_self):
    B, nq, _ = qpos.shape
    ns = sposT.shape[2]
    Tq = min(nq, 128)
    idx = pl.pallas_call(
        functools.partial(_knn_kernel, ns=ns, Tq=Tq, exclude_self=exclude_self),
        grid=(B, nq // Tq),
        in_specs=[pl.BlockSpec((1, Tq, 3), lambda b, t: (b, t, 0)),
                  pl.BlockSpec((1, 3, ns), lambda b, t: (b, 0, 0))],
        out_specs=pl.BlockSpec((1, _K, Tq), lambda b, t: (b, 0, t)),
        out_shape=jax.ShapeDtypeStruct((B, _K, nq), jnp.int32),
    )(qpos, sposT)
    return jnp.transpose(idx, (0, 2, 1)).reshape(-1)


def _sc_gather(table, idx):
    V, D = table.shape
    E = idx.shape[0]
    NC, NS = 2, 16
    NW = NC * NS
    rpw = E // NW
    chunk = min(rpw, 128)
    while chunk * D * 4 > 262144:
        chunk //= 2
    nch = rpw // chunk
    mesh = plsc.VectorSubcoreMesh(core_axis_name="c", subcore_axis_name="s")

    @functools.partial(
        pl.kernel, mesh=mesh,
        out_type=jax.ShapeDtypeStruct((E, D), F32),
        scratch_types=[pltpu.VMEM((chunk,), jnp.int32),
                       pltpu.VMEM((chunk, D), F32),
                       pltpu.SemaphoreType.DMA],
    )
    def k(table_hbm, idx_hbm, out_hbm, idx_v, rows_v, sem):
        wid = lax.axis_index("s") * NC + lax.axis_index("c")
        base = wid * rpw

        def body(j, _):
            off = base + j * chunk
            pltpu.sync_copy(idx_hbm.at[pl.ds(off, chunk)], idx_v)
            pltpu.async_copy(table_hbm.at[idx_v], rows_v, sem).wait()
            pltpu.sync_copy(rows_v, out_hbm.at[pl.ds(off, chunk)])
            return 0

        lax.fori_loop(0, nch, body, 0)

    return k(table, idx)


def _feats_kernel(x_ref, p_ref, win_ref, bin_ref, wl_ref, ws_ref, wd_ref,
                  tbl_ref, ad_ref, *, c):
    x1 = jnp.maximum(_dotT(x_ref[...], win_ref[...]) + bin_ref[...], 0.0)
    tbl_ref[:, 0:c] = _dotT(x1, wl_ref[...])
    tbl_ref[:, c:2 * c] = _dotT(x1, ws_ref[...])
    tbl_ref[:, 2 * c:2 * c + 3] = p_ref[...]
    ad_ref[...] = _dotT(x1, wd_ref[...])


def _edge_kernel(g_ref, tbl_ref, ad_ref, w1p_ref, b1p_ref, w2p_ref, b2p_ref,
                 w1a_ref, b1a_ref, w2a_ref, b2a_ref, wo_ref, bo_ref, o_ref,
                 *, c):
    T = tbl_ref.shape[0]
    g = g_ref[...]
    xv_n = g[:, :, 0:c]
    as_n = g[:, :, c:2 * c]
    ps = g[:, :, 2 * c:2 * c + 3]
    tbl = tbl_ref[...]
    pq = tbl[:, 2 * c:2 * c + 3]
    pd = (pq[:, None, :] - ps).reshape(T * _K, 3)
    h = jnp.maximum(_dotT(pd, w1p_ref[...]) + b1p_ref[...], 0.0)
    delta_n = jnp.maximum(_dotT(h, w2p_ref[...]) + b2p_ref[...], 0.0)
    hs = jnp.maximum(b1p_ref[...], 0.0)
    delta_s = jnp.maximum(_dotT(hs, w2p_ref[...]) + b2p_ref[...], 0.0)
    ad = ad_ref[...]
    ai_n = (ad[:, None, :] - as_n).reshape(T * _K, c) + delta_n
    h2 = jnp.maximum(_dotT(ai_n, w1a_ref[...]) + b1a_ref[...], 0.0)
    al_n = jnp.maximum(_dotT(h2, w2a_ref[...]) + b2a_ref[...], 0.0)
    al_n = al_n.reshape(T, _K, c)
    xv_s = tbl[:, 0:c]
    as_s = tbl[:, c:2 * c]
    ai_s = ad - as_s + delta_s
    h2s = jnp.maximum(_dotT(ai_s, w1a_ref[...]) + b1a_ref[...], 0.0)
    al_s = jnp.maximum(_dotT(h2s, w2a_ref[...]) + b2a_ref[...], 0.0)
    amax = jnp.maximum(jnp.max(al_n, axis=1), al_s)
    en = jnp.exp(al_n - amax[:, None, :])
    es = jnp.exp(al_s - amax)
    den = jnp.sum(en, axis=1) + es + 1e-16
    dn3 = delta_n.reshape(T, _K, c)
    msg = jnp.sum(en * (xv_n + dn3), axis=1) + es * (xv_s + delta_s)
    agg = msg / den
    o_ref[...] = jnp.maximum(_dotT(agg, wo_ref[...]) + bo_ref[...], 0.0)


def _block(bp, x, pos, idx_flat, c):
    n = x.shape[0]
    D = ((2 * c + 16 + 127) // 128) * 128
    T = min(n, 2048)
    G = n // T
    tbl, ad = pl.pallas_call(
        functools.partial(_feats_kernel, c=c),
        grid=(G,),
        in_specs=[pl.BlockSpec((T, c), lambda i: (i, 0)),
                  pl.BlockSpec((T, 3), lambda i: (i, 0)),
                  pl.BlockSpec((c, c), lambda i: (0, 0)),
                  pl.BlockSpec((1, c), lambda i: (0, 0)),
                  pl.BlockSpec((c, c), lambda i: (0, 0)),
                  pl.BlockSpec((c, c), lambda i: (0, 0)),
                  pl.BlockSpec((c, c), lambda i: (0, 0))],
        out_specs=[pl.BlockSpec((T, D), lambda i: (i, 0)),
                   pl.BlockSpec((T, c), lambda i: (i, 0))],
        out_shape=[jax.ShapeDtypeStruct((n, D), F32),
                   jax.ShapeDtypeStruct((n, c), F32)],
    )(x, pos, bp['lin_in']['W'], bp['lin_in']['b'].reshape(1, -1),
      bp['lin']['W'], bp['lin_src']['W'], bp['lin_dst']['W'])

    g = _sc_gather(tbl, idx_flat).reshape(n, _K, D)

    Te = min(n, 128)
    Ge = n // Te
    pn, at = bp['pos_nn'], bp['attn_nn']
    return pl.pallas_call(
        functools.partial(_edge_kernel, c=c),
        grid=(Ge,),
        in_specs=[pl.BlockSpec((Te, _K, D), lambda i: (i, 0, 0)),
                  pl.BlockSpec((Te, D), lambda i: (i, 0)),
                  pl.BlockSpec((Te, c), lambda i: (i, 0)),
                  pl.BlockSpec((64, 3), lambda i: (0, 0)),
                  pl.BlockSpec((1, 64), lambda i: (0, 0)),
                  pl.BlockSpec((c, 64), lambda i: (0, 0)),
                  pl.BlockSpec((1, c), lambda i: (0, 0)),
                  pl.BlockSpec((64, c), lambda i: (0, 0)),
                  pl.BlockSpec((1, 64), lambda i: (0, 0)),
                  pl.BlockSpec((c, 64), lambda i: (0, 0)),
                  pl.BlockSpec((1, c), lambda i: (0, 0)),
                  pl.BlockSpec((c, c), lambda i: (0, 0)),
                  pl.BlockSpec((1, c), lambda i: (0, 0))],
        out_specs=pl.BlockSpec((Te, c), lambda i: (i, 0)),
        out_shape=jax.ShapeDtypeStruct((n, c), F32),
    )(g, tbl, ad,
      pn[0]['W'], pn[0]['b'].reshape(1, -1), pn[1]['W'], pn[1]['b'].reshape(1, -1),
      at[0]['W'], at[0]['b'].reshape(1, -1), at[1]['W'], at[1]['b'].reshape(1, -1),
      bp['lin_out']['W'], bp['lin_out']['b'].reshape(1, -1))


def _max_kernel(g_ref, o_ref, *, c):
    o_ref[...] = jnp.max(g_ref[:, :, 0:c], axis=1)


def _nbr_max(g, c):
    n, _, P = g.shape
    T = min(n, 128)
    return pl.pallas_call(
        functools.partial(_max_kernel, c=c),
        grid=(n // T,),
        in_specs=[pl.BlockSpec((T, _K, P), lambda i: (i, 0, 0))],
        out_specs=pl.BlockSpec((T, c), lambda i: (i, 0)),
        out_shape=jax.ShapeDtypeStruct((n, c), F32),
    )(g)


def _head_kernel(x_ref, w1_ref, b1_ref, w2_ref, b2_ref, o_ref):
    xb = jnp.mean(x_ref[...], axis=1)
    h = jnp.maximum(_dotT(xb, w1_ref[...]) + b1_ref[...], 0.0)
    o_ref[...] = _dotT(h, w2_ref[...]) + b2_ref[...]


def _head(x, ps):
    B, n, c = x.shape
    return pl.pallas_call(
        _head_kernel,
        grid=(1,),
        in_specs=[pl.BlockSpec((B, n, c), lambda i: (0, 0, 0)),
                  pl.BlockSpec((64, c), lambda i: (0, 0)),
                  pl.BlockSpec((1, 64), lambda i: (0, 0)),
                  pl.BlockSpec((6, 64), lambda i: (0, 0)),
                  pl.BlockSpec((1, 6), lambda i: (0, 0))],
        out_specs=pl.BlockSpec((B, 6), lambda i: (0, 0)),
        out_shape=jax.ShapeDtypeStruct((B, 6), F32),
    )(x, ps[0]['W'], ps[0]['b'].reshape(1, -1),
      ps[1]['W'], ps[1]['b'].reshape(1, -1))


def kernel(data, params):
    B, N, C = data.shape
    pos = data[..., :3]
    posT = jnp.transpose(pos, (0, 2, 1))

    x = _lin_bn_relu(data.reshape(B * N, C),
                     params['mlp_input']['lin'], params['mlp_input']['bn'])
    idx_flat = _knn(pos, posT, exclude_self=True)
    x = _block(params['tf_input'], x, pos.reshape(-1, 3), idx_flat, _DIMS[0])

    cur_pos, cur_posT, n_l = pos, posT, N
    for i in range(4):
        m = int(np.ceil(0.25 * n_l))
        sub = _fps(cur_posT, m)
        gidx = _knn(sub, cur_posT, exclude_self=False)
        cout = params['td'][i]['lin']['W'].shape[0]
        P = ((cout + 127) // 128) * 128
        y = _lin_bn_relu(x, params['td'][i]['lin'], params['td'][i]['bn'],
                         pad_to=P)
        x = _nbr_max(_sc_gather(y, gidx).reshape(B * m, _K, P), cout)
        cur_pos, cur_posT, n_l = sub, jnp.transpose(sub, (0, 2, 1)), m
        idx_flat = _knn(cur_pos, cur_posT, exclude_self=True)
        x = _block(params['tf'][i], x, cur_pos.reshape(-1, 3), idx_flat,
                   _DIMS[i + 1])

    return _head(x.reshape(B, n_l, -1), params['mlp_out'])


# speedup vs baseline: 8.9983x; 1.2487x over previous
import functools

import numpy as np
import jax
import jax.numpy as jnp
from jax import lax
from jax.experimental import pallas as pl
from jax.experimental.pallas import tpu as pltpu
from jax.experimental.pallas import tpu_sc as plsc

F32 = jnp.float32
_K = 16
_HI = lax.Precision.HIGHEST
_DIMS = (32, 64, 128, 256, 512)


def _dotT(x, w):
    return lax.dot_general(x, w, (((1,), (1,)), ((), ())),
                           precision=_HI, preferred_element_type=F32)


def _dot(x, w):
    return lax.dot_general(x, w, (((1,), (0,)), ((), ())),
                           precision=_HI, preferred_element_type=F32)


def _mm_kernel(x_ref, w_ref, b_ref, y_ref, s1_ref, s2_ref):
    y = _dotT(x_ref[...], w_ref[...]) + b_ref[...]
    y_ref[...] = y
    s1_ref[0] = jnp.sum(y, axis=0, keepdims=True)
    s2_ref[0] = jnp.sum(y * y, axis=0, keepdims=True)


def _bn_kernel(y_ref, s1_ref, s2_ref, g_ref, b_ref, o_ref, *, n, cout):
    mu = jnp.sum(s1_ref[...], axis=0) / n
    ex2 = jnp.sum(s2_ref[...], axis=0) / n
    var = ex2 - mu * mu
    y = y_ref[...]
    o_ref[:, 0:cout] = jnp.maximum(
        (y - mu) / jnp.sqrt(var + 1e-5) * g_ref[...] + b_ref[...], 0.0)


def _lin_bn_relu(x, lin, bn, pad_to=None):
    n, cin = x.shape
    cout = lin['W'].shape[0]
    P = pad_to if pad_to is not None else cout
    T = min(n, 2048)
    G = n // T
    y, s1, s2 = pl.pallas_call(
        _mm_kernel,
        grid=(G,),
        in_specs=[pl.BlockSpec((T, cin), lambda i: (i, 0)),
                  pl.BlockSpec((cout, cin), lambda i: (0, 0)),
                  pl.BlockSpec((1, cout), lambda i: (0, 0))],
        out_specs=[pl.BlockSpec((T, cout), lambda i: (i, 0)),
                   pl.BlockSpec((1, 1, cout), lambda i: (i, 0, 0)),
                   pl.BlockSpec((1, 1, cout), lambda i: (i, 0, 0))],
        out_shape=[jax.ShapeDtypeStruct((n, cout), F32),
                   jax.ShapeDtypeStruct((G, 1, cout), F32),
                   jax.ShapeDtypeStruct((G, 1, cout), F32)],
    )(x, lin['W'], lin['b'].reshape(1, -1))
    return pl.pallas_call(
        functools.partial(_bn_kernel, n=float(n), cout=cout),
        grid=(G,),
        in_specs=[pl.BlockSpec((T, cout), lambda i: (i, 0)),
                  pl.BlockSpec((G, 1, cout), lambda i: (0, 0, 0)),
                  pl.BlockSpec((G, 1, cout), lambda i: (0, 0, 0)),
                  pl.BlockSpec((1, cout), lambda i: (0, 0)),
                  pl.BlockSpec((1, cout), lambda i: (0, 0))],
        out_specs=pl.BlockSpec((T, P), lambda i: (i, 0)),
        out_shape=jax.ShapeDtypeStruct((n, P), F32),
    )(y, s1, s2, bn['gamma'].reshape(1, -1), bn['beta'].reshape(1, -1))


def _fps_kernel(pt_ref, sub_ref, *, m, n):
    B = pt_ref.shape[0]
    pt = pt_ref[...]
    for j in range(3):
        sub_ref[:, 0:1, j:j + 1] = pt[:, j:j + 1, 0:1]
    iota = lax.broadcasted_iota(jnp.int32, (B, n), 1)

    def body(i, carry):
        dists, last = carry
        d = jnp.sum((pt - last) ** 2, axis=1)
        dists = jnp.minimum(dists, d)
        mx = jnp.max(dists, axis=1, keepdims=True)
        idx = jnp.min(jnp.where(dists == mx, iota, n), axis=1, keepdims=True)
        mask = (iota == idx)[:, None, :]
        lastc = jnp.sum(jnp.where(mask, pt, 0.0), axis=2, keepdims=True)
        for j in range(3):
            sub_ref[:, pl.ds(i, 1), j:j + 1] = lastc[:, j:j + 1, :]
        return dists, lastc

    init = (jnp.full((B, n), jnp.inf, F32), pt[:, :, 0:1])
    lax.fori_loop(1, m, body, init)


def _fps(posT, m):
    B, _, n = posT.shape
    return pl.pallas_call(
        functools.partial(_fps_kernel, m=m, n=n),
        grid=(1,),
        in_specs=[pl.BlockSpec((B, 3, n), lambda i: (0, 0, 0))],
        out_specs=pl.BlockSpec((B, m, 3), lambda i: (0, 0, 0)),
        out_shape=jax.ShapeDtypeStruct((B, m, 3), F32),
    )(posT)


def _knn_kernel(q_ref, st_ref, o_ref, d_ref, *, ns, Tq, exclude_self):
    b = pl.program_id(0)
    t = pl.program_id(1)
    q = q_ref[0]
    st = st_ref[0]
    sq_q = jnp.sum(q * q, axis=1, keepdims=True)
    sq_s = jnp.sum(st * st, axis=0, keepdims=True)
    mm = lax.dot_general(q.astype(jnp.bfloat16), st.astype(jnp.bfloat16),
                         (((1,), (0,)), ((), ())),
                         preferred_element_type=F32)
    d = sq_q + sq_s - 2.0 * mm
    ci = lax.broadcasted_iota(jnp.int32, (Tq, ns), 1)
    if exclude_self:
        qid = t * Tq + lax.broadcasted_iota(jnp.int32, (Tq, ns), 0)
        d = jnp.where(ci == qid, jnp.inf, d)
    off = b * ns
    d_ref[...] = d

    def step(k, _):
        idx = jnp.argmin(d_ref[...], axis=1).astype(jnp.int32)
        o_ref[0, pl.ds(k, 1), :] = (idx + off)[None, :]
        d_ref[...] = jnp.where(ci == idx[:, None], jnp.inf, d_ref[...])
        return 0

    lax.fori_loop(0, _K, step, 0)


def _knn(qpos, sposT, exclude_self):
    B, nq, _ = qpos.shape
    ns = sposT.shape[2]
    Tq = min(nq, 128)
    idx = pl.pallas_call(
        functools.partial(_knn_kernel, ns=ns, Tq=Tq, exclude_self=exclude_self),
        grid=(B, nq // Tq),
        in_specs=[pl.BlockSpec((1, Tq, 3), lambda b, t: (b, t, 0)),
                  pl.BlockSpec((1, 3, ns), lambda b, t: (b, 0, 0))],
        out_specs=pl.BlockSpec((1, _K, Tq), lambda b, t: (b, 0, t)),
        out_shape=jax.ShapeDtypeStruct((B, _K, nq), jnp.int32),
        scratch_shapes=[pltpu.VMEM((Tq, ns), F32)],
    )(qpos, sposT)
    return jnp.transpose(idx, (0, 2, 1)).reshape(-1)


def _sc_gather(table, idx):
    V, D = table.shape
    E = idx.shape[0]
    NC, NS = 2, 16
    NW = NC * NS
    rpw = E // NW
    chunk = min(rpw, 128)
    while chunk * D * 4 > 262144:
        chunk //= 2
    nch = rpw // chunk
    mesh = plsc.VectorSubcoreMesh(core_axis_name="c", subcore_axis_name="s")

    @functools.partial(
        pl.kernel, mesh=mesh,
        out_type=jax.ShapeDtypeStruct((E, D), F32),
        scratch_types=[pltpu.VMEM((chunk,), jnp.int32),
                       pltpu.VMEM((chunk, D), F32),
                       pltpu.SemaphoreType.DMA],
    )
    def k(table_hbm, idx_hbm, out_hbm, idx_v, rows_v, sem):
        wid = lax.axis_index("s") * NC + lax.axis_index("c")
        base = wid * rpw

        def body(j, _):
            off = base + j * chunk
            pltpu.sync_copy(idx_hbm.at[pl.ds(off, chunk)], idx_v)
            pltpu.async_copy(table_hbm.at[idx_v], rows_v, sem).wait()
            pltpu.sync_copy(rows_v, out_hbm.at[pl.ds(off, chunk)])
            return 0

        lax.fori_loop(0, nch, body, 0)

    return k(table, idx)


def _feats_kernel(x_ref, p_ref, win_ref, bin_ref, wl_ref, ws_ref, wd_ref,
                  tbl_ref, ad_ref, *, c):
    x1 = jnp.maximum(_dotT(x_ref[...], win_ref[...]) + bin_ref[...], 0.0)
    tbl_ref[:, 0:c] = _dotT(x1, wl_ref[...])
    tbl_ref[:, c:2 * c] = _dotT(x1, ws_ref[...])
    tbl_ref[:, 2 * c:2 * c + 3] = p_ref[...]
    ad_ref[...] = _dotT(x1, wd_ref[...])


def _edge_kernel(g_ref, tbl_ref, ad_ref, w1p_ref, b1p_ref, w2p_ref, b2p_ref,
                 w1a_ref, b1a_ref, w2a_ref, b2a_ref, wo_ref, bo_ref, o_ref,
                 *, c):
    T = tbl_ref.shape[0]
    g = g_ref[...]
    xv_n = g[:, :, 0:c]
    as_n = g[:, :, c:2 * c]
    ps = g[:, :, 2 * c:2 * c + 3]
    tbl = tbl_ref[...]
    pq = tbl[:, 2 * c:2 * c + 3]
    pd = (pq[:, None, :] - ps).reshape(T * _K, 3)
    h = jnp.maximum(_dotT(pd, w1p_ref[...]) + b1p_ref[...], 0.0)
    delta_n = jnp.maximum(_dotT(h, w2p_ref[...]) + b2p_ref[...], 0.0)
    hs = jnp.maximum(b1p_ref[...], 0.0)
    delta_s = jnp.maximum(_dotT(hs, w2p_ref[...]) + b2p_ref[...], 0.0)
    ad = ad_ref[...]
    ai_n = (ad[:, None, :] - as_n).reshape(T * _K, c) + delta_n
    h2 = jnp.maximum(_dotT(ai_n, w1a_ref[...]) + b1a_ref[...], 0.0)
    al_n = jnp.maximum(_dotT(h2, w2a_ref[...]) + b2a_ref[...], 0.0)
    al_n = al_n.reshape(T, _K, c)
    xv_s = tbl[:, 0:c]
    as_s = tbl[:, c:2 * c]
    ai_s = ad - as_s + delta_s
    h2s = jnp.maximum(_dotT(ai_s, w1a_ref[...]) + b1a_ref[...], 0.0)
    al_s = jnp.maximum(_dotT(h2s, w2a_ref[...]) + b2a_ref[...], 0.0)
    amax = jnp.maximum(jnp.max(al_n, axis=1), al_s)
    en = jnp.exp(al_n - amax[:, None, :])
    es = jnp.exp(al_s - amax)
    den = jnp.sum(en, axis=1) + es + 1e-16
    dn3 = delta_n.reshape(T, _K, c)
    msg = jnp.sum(en * (xv_n + dn3), axis=1) + es * (xv_s + delta_s)
    agg = msg / den
    o_ref[...] = jnp.maximum(_dotT(agg, wo_ref[...]) + bo_ref[...], 0.0)


def _block(bp, x, pos, idx_flat, c):
    n = x.shape[0]
    D = ((2 * c + 16 + 127) // 128) * 128
    T = min(n, 2048)
    G = n // T
    tbl, ad = pl.pallas_call(
        functools.partial(_feats_kernel, c=c),
        grid=(G,),
        in_specs=[pl.BlockSpec((T, c), lambda i: (i, 0)),
                  pl.BlockSpec((T, 3), lambda i: (i, 0)),
                  pl.BlockSpec((c, c), lambda i: (0, 0)),
                  pl.BlockSpec((1, c), lambda i: (0, 0)),
                  pl.BlockSpec((c, c), lambda i: (0, 0)),
                  pl.BlockSpec((c, c), lambda i: (0, 0)),
                  pl.BlockSpec((c, c), lambda i: (0, 0))],
        out_specs=[pl.BlockSpec((T, D), lambda i: (i, 0)),
                   pl.BlockSpec((T, c), lambda i: (i, 0))],
        out_shape=[jax.ShapeDtypeStruct((n, D), F32),
                   jax.ShapeDtypeStruct((n, c), F32)],
    )(x, pos, bp['lin_in']['W'], bp['lin_in']['b'].reshape(1, -1),
      bp['lin']['W'], bp['lin_src']['W'], bp['lin_dst']['W'])

    g = _sc_gather(tbl, idx_flat).reshape(n, _K, D)

    Te = min(n, 128)
    Ge = n // Te
    pn, at = bp['pos_nn'], bp['attn_nn']
    return pl.pallas_call(
        functools.partial(_edge_kernel, c=c),
        grid=(Ge,),
        in_specs=[pl.BlockSpec((Te, _K, D), lambda i: (i, 0, 0)),
                  pl.BlockSpec((Te, D), lambda i: (i, 0)),
                  pl.BlockSpec((Te, c), lambda i: (i, 0)),
                  pl.BlockSpec((64, 3), lambda i: (0, 0)),
                  pl.BlockSpec((1, 64), lambda i: (0, 0)),
                  pl.BlockSpec((c, 64), lambda i: (0, 0)),
                  pl.BlockSpec((1, c), lambda i: (0, 0)),
                  pl.BlockSpec((64, c), lambda i: (0, 0)),
                  pl.BlockSpec((1, 64), lambda i: (0, 0)),
                  pl.BlockSpec((c, 64), lambda i: (0, 0)),
                  pl.BlockSpec((1, c), lambda i: (0, 0)),
                  pl.BlockSpec((c, c), lambda i: (0, 0)),
                  pl.BlockSpec((1, c), lambda i: (0, 0))],
        out_specs=pl.BlockSpec((Te, c), lambda i: (i, 0)),
        out_shape=jax.ShapeDtypeStruct((n, c), F32),
    )(g, tbl, ad,
      pn[0]['W'], pn[0]['b'].reshape(1, -1), pn[1]['W'], pn[1]['b'].reshape(1, -1),
      at[0]['W'], at[0]['b'].reshape(1, -1), at[1]['W'], at[1]['b'].reshape(1, -1),
      bp['lin_out']['W'], bp['lin_out']['b'].reshape(1, -1))


def _max_kernel(g_ref, o_ref, *, c):
    o_ref[...] = jnp.max(g_ref[:, :, 0:c], axis=1)


def _nbr_max(g, c):
    n, _, P = g.shape
    T = min(n, 128)
    return pl.pallas_call(
        functools.partial(_max_kernel, c=c),
        grid=(n // T,),
        in_specs=[pl.BlockSpec((T, _K, P), lambda i: (i, 0, 0))],
        out_specs=pl.BlockSpec((T, c), lambda i: (i, 0)),
        out_shape=jax.ShapeDtypeStruct((n, c), F32),
    )(g)


def _head_kernel(x_ref, w1_ref, b1_ref, w2_ref, b2_ref, o_ref):
    xb = jnp.mean(x_ref[...], axis=1)
    h = jnp.maximum(_dotT(xb, w1_ref[...]) + b1_ref[...], 0.0)
    o_ref[...] = _dotT(h, w2_ref[...]) + b2_ref[...]


def _head(x, ps):
    B, n, c = x.shape
    return pl.pallas_call(
        _head_kernel,
        grid=(1,),
        in_specs=[pl.BlockSpec((B, n, c), lambda i: (0, 0, 0)),
                  pl.BlockSpec((64, c), lambda i: (0, 0)),
                  pl.BlockSpec((1, 64), lambda i: (0, 0)),
                  pl.BlockSpec((6, 64), lambda i: (0, 0)),
                  pl.BlockSpec((1, 6), lambda i: (0, 0))],
        out_specs=pl.BlockSpec((B, 6), lambda i: (0, 0)),
        out_shape=jax.ShapeDtypeStruct((B, 6), F32),
    )(x, ps[0]['W'], ps[0]['b'].reshape(1, -1),
      ps[1]['W'], ps[1]['b'].reshape(1, -1))


def kernel(data, params):
    B, N, C = data.shape
    pos = data[..., :3]
    posT = jnp.transpose(pos, (0, 2, 1))

    x = _lin_bn_relu(data.reshape(B * N, C),
                     params['mlp_input']['lin'], params['mlp_input']['bn'])
    idx_flat = _knn(pos, posT, exclude_self=True)
    x = _block(params['tf_input'], x, pos.reshape(-1, 3), idx_flat, _DIMS[0])

    cur_pos, cur_posT, n_l = pos, posT, N
    for i in range(4):
        m = int(np.ceil(0.25 * n_l))
        sub = _fps(cur_posT, m)
        gidx = _knn(sub, cur_posT, exclude_self=False)
        cout = params['td'][i]['lin']['W'].shape[0]
        P = ((cout + 127) // 128) * 128
        y = _lin_bn_relu(x, params['td'][i]['lin'], params['td'][i]['bn'],
                         pad_to=P)
        x = _nbr_max(_sc_gather(y, gidx).reshape(B * m, _K, P), cout)
        cur_pos, cur_posT, n_l = sub, jnp.transpose(sub, (0, 2, 1)), m
        idx_flat = _knn(cur_pos, cur_posT, exclude_self=True)
        x = _block(params['tf'][i], x, cur_pos.reshape(-1, 3), idx_flat,
                   _DIMS[i + 1])

    return _head(x.reshape(B, n_l, -1), params['mlp_out'])
